# Initial kernel scaffold; baseline (speedup 1.0000x reference)
import jax
import jax.numpy as jnp
from jax.experimental import pallas as pl


def kernel(input_index, embeds, graph):
    raise NotImplementedError("write your pallas kernel here")


# speedup vs baseline: 59.1037x; 59.1037x over previous
import jax
import jax.numpy as jnp
from jax import lax
from jax.experimental import pallas as pl
from jax.experimental.pallas import tpu as pltpu
from jax.experimental.pallas import tpu_sc as plsc

NUM_POINTS = 4096
DIMS = 128
BATCH = 512

_NC = 2
_NS = 16
_NW = _NC * _NS
_BPW = BATCH // _NW
_L = 16


def _sc_gather_body(idx_hbm, embeds_hbm, graph_hbm, g_out, x_out,
                    idx_all, idx_my, rows, xrows, gcols,
                    sem_g, sem_x):
    c = lax.axis_index("c")
    s = lax.axis_index("s")
    wid = s * _NC + c
    base = wid * _BPW
    pltpu.sync_copy(idx_hbm, idx_all)
    pltpu.sync_copy(idx_hbm.at[pl.ds(base, _BPW)], idx_my)
    cp_g = pltpu.async_copy(graph_hbm.at[idx_my], rows, sem_g)
    cp_x = pltpu.async_copy(embeds_hbm.at[idx_my], xrows, sem_x)
    cp_x.wait()
    pltpu.sync_copy(xrows, x_out.at[pl.ds(base, _BPW)])
    cp_g.wait()

    for li in range(_BPW):
        rsel = jnp.full((_L,), li, jnp.int32)

        def col_block(jb, carry, rsel=rsel, li=li):
            cols = idx_all[pl.ds(jb * _L, _L)]
            vals = plsc.load_gather(rows, [rsel, cols])
            gcols[pl.ds(li * BATCH + jb * _L, _L)] = vals
            return carry

        lax.fori_loop(0, BATCH // _L, col_block, 0)

    pltpu.sync_copy(gcols, g_out.at[pl.ds(base * BATCH, _BPW * BATCH)])


_sc_gather = pl.kernel(
    _sc_gather_body,
    out_type=[
        jax.ShapeDtypeStruct((BATCH * BATCH,), jnp.float32),
        jax.ShapeDtypeStruct((BATCH, DIMS), jnp.float32),
    ],
    mesh=plsc.VectorSubcoreMesh(core_axis_name="c", subcore_axis_name="s"),
    compiler_params=pltpu.CompilerParams(needs_layout_passes=False),
    scratch_types=[
        pltpu.VMEM((BATCH,), jnp.int32),
        pltpu.VMEM((_BPW,), jnp.int32),
        pltpu.VMEM((_BPW, NUM_POINTS), jnp.float32),
        pltpu.VMEM((_BPW, DIMS), jnp.float32),
        pltpu.VMEM((_BPW * BATCH,), jnp.float32),
        pltpu.SemaphoreType.DMA,
        pltpu.SemaphoreType.DMA,
    ],
)


def _tc_loss_body(x_ref, g_ref, o_ref):
    x = x_ref[...]
    g = g_ref[...]
    xx = lax.dot_general(x, x, (((1,), (1,)), ((), ())),
                         preferred_element_type=jnp.float32)
    n2 = jnp.sum(x * x, axis=1)
    d2 = n2[:, None] + n2[None, :] - 2.0 * xx + 1e-12
    m = jnp.abs(d2 / (g * g) - 1.0)
    row = lax.broadcasted_iota(jnp.int32, (BATCH, BATCH), 0)
    col = lax.broadcasted_iota(jnp.int32, (BATCH, BATCH), 1)
    o_ref[0, 0] = jnp.sum(jnp.where(row < col, m, 0.0))


_tc_loss = pl.pallas_call(
    _tc_loss_body,
    out_shape=jax.ShapeDtypeStruct((1, 1), jnp.float32),
    out_specs=pl.BlockSpec(memory_space=pltpu.SMEM),
)


def kernel(input_index, embeds, graph):
    idx = input_index.astype(jnp.int32)
    gflat, x = _sc_gather(idx, embeds, graph)
    return _tc_loss(x, gflat.reshape(BATCH, BATCH))[0, 0]


# speedup vs baseline: 69.1364x; 1.1697x over previous
import jax
import jax.numpy as jnp
from jax import lax
from jax.experimental import pallas as pl
from jax.experimental.pallas import tpu as pltpu
from jax.experimental.pallas import tpu_sc as plsc

NUM_POINTS = 4096
DIMS = 128
BATCH = 512

_NC = 2
_NS = 16
_NW = _NC * _NS
_BPW = BATCH // _NW
_L = 16


def _sc_gather_body(idx_hbm, embeds_hbm, graph_hbm, g_out, x_out,
                    idx_all, idx_my, rows, xrows, gcols,
                    sem_g, sem_x):
    c = lax.axis_index("c")
    s = lax.axis_index("s")
    wid = s * _NC + c
    base = wid * _BPW
    pltpu.sync_copy(idx_hbm, idx_all)
    pltpu.sync_copy(idx_hbm.at[pl.ds(base, _BPW)], idx_my)
    cp_g = pltpu.async_copy(graph_hbm.at[idx_my], rows, sem_g)
    cp_x = pltpu.async_copy(embeds_hbm.at[idx_my], xrows, sem_x)
    cp_x.wait()
    pltpu.sync_copy(xrows, x_out.at[pl.ds(base, _BPW)])
    cp_g.wait()

    def col_block(jb, carry):
        cols = idx_all[pl.ds(jb * _L, _L)]
        for li in range(_BPW):
            rsel = jnp.full((_L,), li, jnp.int32)
            vals = plsc.load_gather(rows, [rsel, cols])
            gcols[pl.ds(li * BATCH + jb * _L, _L)] = vals
        return carry

    lax.fori_loop(0, BATCH // _L, col_block, 0)

    for li in range(_BPW):
        pltpu.sync_copy(gcols.at[pl.ds(li * BATCH, BATCH)],
                        g_out.at[base + li])


_sc_gather = pl.kernel(
    _sc_gather_body,
    out_type=[
        jax.ShapeDtypeStruct((BATCH, BATCH), jnp.float32),
        jax.ShapeDtypeStruct((BATCH, DIMS), jnp.float32),
    ],
    mesh=plsc.VectorSubcoreMesh(core_axis_name="c", subcore_axis_name="s"),
    compiler_params=pltpu.CompilerParams(needs_layout_passes=False),
    scratch_types=[
        pltpu.VMEM((BATCH,), jnp.int32),
        pltpu.VMEM((_BPW,), jnp.int32),
        pltpu.VMEM((_BPW, NUM_POINTS), jnp.float32),
        pltpu.VMEM((_BPW, DIMS), jnp.float32),
        pltpu.VMEM((_BPW * BATCH,), jnp.float32),
        pltpu.SemaphoreType.DMA,
        pltpu.SemaphoreType.DMA,
    ],
)


def _tc_loss_body(x_ref, g_ref, o_ref):
    x = x_ref[...]
    g = g_ref[...]
    xx = lax.dot_general(x, x, (((1,), (1,)), ((), ())),
                         preferred_element_type=jnp.float32)
    n2 = jnp.sum(x * x, axis=1)
    d2 = n2[:, None] + n2[None, :] - 2.0 * xx + 1e-12
    m = jnp.abs(d2 / (g * g) - 1.0)
    row = lax.broadcasted_iota(jnp.int32, (BATCH, BATCH), 0)
    col = lax.broadcasted_iota(jnp.int32, (BATCH, BATCH), 1)
    o_ref[0, 0] = jnp.sum(jnp.where(row < col, m, 0.0))


_tc_loss = pl.pallas_call(
    _tc_loss_body,
    out_shape=jax.ShapeDtypeStruct((1, 1), jnp.float32),
    out_specs=pl.BlockSpec(memory_space=pltpu.SMEM),
)


def kernel(input_index, embeds, graph):
    idx = input_index.astype(jnp.int32)
    gmat, x = _sc_gather(idx, embeds, graph)
    return _tc_loss(x, gmat)[0, 0]


# speedup vs baseline: 69.7958x; 1.0095x over previous
import jax
import jax.numpy as jnp
from jax import lax
from jax.experimental import pallas as pl
from jax.experimental.pallas import tpu as pltpu
from jax.experimental.pallas import tpu_sc as plsc

NUM_POINTS = 4096
DIMS = 128
BATCH = 512

_NC = 2
_NS = 16
_NW = _NC * _NS
_BPW = BATCH // _NW
_L = 16


def _sc_gather_body(idx_hbm, embeds_hbm, graph_hbm, g_out, x_out,
                    idx_all, idx_my, rows, xrows, gcols,
                    sem_g, sem_g2, sem_x):
    c = lax.axis_index("c")
    s = lax.axis_index("s")
    wid = s * _NC + c
    base = wid * _BPW
    half = _BPW // 2
    pltpu.sync_copy(idx_hbm, idx_all)
    pltpu.sync_copy(idx_hbm.at[pl.ds(base, _BPW)], idx_my)
    cp_g0 = pltpu.async_copy(graph_hbm.at[idx_my.at[pl.ds(0, half)]],
                             rows.at[pl.ds(0, half)], sem_g)
    cp_g1 = pltpu.async_copy(graph_hbm.at[idx_my.at[pl.ds(half, half)]],
                             rows.at[pl.ds(half, half)], sem_g2)
    cp_x = pltpu.async_copy(embeds_hbm.at[idx_my], xrows, sem_x)

    def make_col_block(lo):
        def col_block(jb, carry):
            cols = idx_all[pl.ds(jb * _L, _L)]
            for li in range(lo, lo + half):
                rsel = jnp.full((_L,), li, jnp.int32)
                vals = plsc.load_gather(rows, [rsel, cols])
                gcols[li, pl.ds(jb * _L, _L)] = vals
            return carry
        return col_block

    cp_g0.wait()
    lax.fori_loop(0, BATCH // _L, make_col_block(0), 0)
    cp_g1.wait()
    lax.fori_loop(0, BATCH // _L, make_col_block(half), 0)

    pltpu.sync_copy(gcols, g_out.at[pl.ds(base, _BPW)])
    cp_x.wait()
    pltpu.sync_copy(xrows, x_out.at[pl.ds(base, _BPW)])


_sc_gather = pl.kernel(
    _sc_gather_body,
    out_type=[
        jax.ShapeDtypeStruct((BATCH, BATCH), jnp.float32),
        jax.ShapeDtypeStruct((BATCH, DIMS), jnp.float32),
    ],
    mesh=plsc.VectorSubcoreMesh(core_axis_name="c", subcore_axis_name="s"),
    compiler_params=pltpu.CompilerParams(needs_layout_passes=False),
    scratch_types=[
        pltpu.VMEM((BATCH,), jnp.int32),
        pltpu.VMEM((_BPW,), jnp.int32),
        pltpu.VMEM((_BPW, NUM_POINTS), jnp.float32),
        pltpu.VMEM((_BPW, DIMS), jnp.float32),
        pltpu.VMEM((_BPW, BATCH), jnp.float32),
        pltpu.SemaphoreType.DMA,
        pltpu.SemaphoreType.DMA,
        pltpu.SemaphoreType.DMA,
    ],
)


def _tc_loss_body(x_ref, g_ref, o_ref):
    x = x_ref[...]
    g = g_ref[...]
    xx = lax.dot_general(x, x, (((1,), (1,)), ((), ())),
                         preferred_element_type=jnp.float32)
    n2 = jnp.sum(x * x, axis=1)
    d2 = n2[:, None] + n2[None, :] - 2.0 * xx + 1e-12
    m = jnp.abs(d2 / (g * g) - 1.0)
    row = lax.broadcasted_iota(jnp.int32, (BATCH, BATCH), 0)
    col = lax.broadcasted_iota(jnp.int32, (BATCH, BATCH), 1)
    o_ref[0, 0] = jnp.sum(jnp.where(row < col, m, 0.0))


_tc_loss = pl.pallas_call(
    _tc_loss_body,
    out_shape=jax.ShapeDtypeStruct((1, 1), jnp.float32),
    out_specs=pl.BlockSpec(memory_space=pltpu.SMEM),
)


def kernel(input_index, embeds, graph):
    idx = input_index.astype(jnp.int32)
    gmat, x = _sc_gather(idx, embeds, graph)
    return _tc_loss(x, gmat)[0, 0]
